# Initial kernel scaffold; baseline (speedup 1.0000x reference)
import jax
import jax.numpy as jnp
from jax.experimental import pallas as pl


def kernel(x, emb, W1, b1, W2, b2, Wout, bout):
    raise NotImplementedError("write your pallas kernel here")


# speedup vs baseline: 34.6356x; 34.6356x over previous
import functools

import jax
import jax.numpy as jnp
from jax import lax
from jax.experimental import pallas as pl
from jax.experimental.pallas import tpu as pltpu
from jax.experimental.pallas import tpu_sc as plsc

BATCH = 16384
F = 100
D = 16
NC, NS = 2, 16
NW = NC * NS
PER_W = BATCH // NW
CB = 16
NCH = PER_W // CB
FLAT = CB * F


def _sc_pool_body(x_hbm, emb_hbm, out_hbm, idx_v, rows_v, out_v, sem):
    c = lax.axis_index("c")
    s = lax.axis_index("s")
    wid = s * NC + c

    def chunk(i, carry):
        base = wid * PER_W + i * CB
        fbase = base * F
        pltpu.sync_copy(x_hbm.at[pl.ds(fbase, FLAT)], idx_v)
        pltpu.async_copy(emb_hbm.at[idx_v], rows_v, sem).wait()
        for b in range(CB):
            def facc(j, accs, b=b):
                a0, a1, a2, a3 = accs
                r = b * F + j * 4
                a0 = a0 + rows_v[r, :]
                a1 = a1 + rows_v[r + 1, :]
                a2 = a2 + rows_v[r + 2, :]
                a3 = a3 + rows_v[r + 3, :]
                return (a0, a1, a2, a3)
            z = jnp.zeros((D,), jnp.float32)
            a0, a1, a2, a3 = lax.fori_loop(0, F // 4, facc, (z, z, z, z))
            out_v[b, :] = (a0 + a1) + (a2 + a3)
        pltpu.sync_copy(out_v, out_hbm.at[pl.ds(base, CB)])
        return carry

    lax.fori_loop(0, NCH, chunk, 0)


_sc_pool = functools.partial(
    pl.kernel,
    out_type=jax.ShapeDtypeStruct((BATCH, D), jnp.float32),
    mesh=plsc.VectorSubcoreMesh(core_axis_name="c", subcore_axis_name="s"),
    scratch_types=[
        pltpu.VMEM((FLAT,), jnp.int32),
        pltpu.VMEM((FLAT, D), jnp.float32),
        pltpu.VMEM((CB, D), jnp.float32),
        pltpu.SemaphoreType.DMA,
    ],
    compiler_params=pltpu.CompilerParams(use_tc_tiling_on_sc=False),
)(_sc_pool_body)


BLK = 2048


def _mlp_body(h_ref, w1_ref, b1_ref, w2_ref, b2_ref, wo_ref, bo_ref, o_ref):
    h = h_ref[...]
    z = jnp.dot(h, w1_ref[...], preferred_element_type=jnp.float32) + b1_ref[...]
    z = jnp.dot(z, w2_ref[...], preferred_element_type=jnp.float32) + b2_ref[...]
    z = jnp.dot(z, wo_ref[...], preferred_element_type=jnp.float32) + bo_ref[...]
    o_ref[...] = jax.nn.sigmoid(z)


def _mlp(pooled, W1, b1, W2, b2, Wout, bout):
    h1, h2, h3 = W1.shape[0], W2.shape[0], Wout.shape[0]
    ncls = Wout.shape[1]
    full = lambda shape: pl.BlockSpec(shape, lambda i: (0, 0))
    return pl.pallas_call(
        _mlp_body,
        grid=(BATCH // BLK,),
        in_specs=[
            pl.BlockSpec((BLK, h1), lambda i: (i, 0)),
            full((h1, h2)), full((1, h2)),
            full((h2, h3)), full((1, h3)),
            full((h3, ncls)), full((1, ncls)),
        ],
        out_specs=pl.BlockSpec((BLK, ncls), lambda i: (i, 0)),
        out_shape=jax.ShapeDtypeStruct((BATCH, ncls), jnp.float32),
    )(pooled, W1, b1.reshape(1, -1), W2, b2.reshape(1, -1),
      Wout, bout.reshape(1, -1))


def kernel(x, emb, W1, b1, W2, b2, Wout, bout):
    pooled = _sc_pool(x.reshape(-1), emb)
    return _mlp(pooled, W1, b1, W2, b2, Wout, bout)


# speedup vs baseline: 49.3385x; 1.4245x over previous
import functools

import jax
import jax.numpy as jnp
from jax import lax
from jax.experimental import pallas as pl
from jax.experimental.pallas import tpu as pltpu
from jax.experimental.pallas import tpu_sc as plsc

BATCH = 16384
F = 100
D = 16
NC, NS = 2, 16
NW = NC * NS
PER_W = BATCH // NW
CB = 32
NCH = PER_W // CB
FLAT = CB * F


def _sc_pool_body(x_hbm, emb_hbm, out_hbm, idx_v, rows_v, out_v,
                  gsem, isem, osem):
    c = lax.axis_index("c")
    s = lax.axis_index("s")
    wid = s * NC + c
    row0 = wid * PER_W

    def idx_copy(i, buf):
        return pltpu.make_async_copy(
            x_hbm.at[pl.ds((row0 + i * CB) * F, FLAT)], idx_v.at[buf],
            isem.at[buf])

    def gather(i, buf):
        return pltpu.make_async_copy(
            emb_hbm.at[idx_v.at[buf]], rows_v.at[buf], gsem.at[buf])

    def out_copy(i, buf):
        return pltpu.make_async_copy(
            out_v.at[buf], out_hbm.at[pl.ds(row0 + i * CB, CB)], osem.at[buf])

    pltpu.sync_copy(x_hbm.at[pl.ds(row0 * F, FLAT)], idx_v.at[0])
    gather(0, 0).start()
    idx_copy(1, 1).start()

    def outer(i0, carry):
        for b in range(2):
            i = i0 * 2 + b
            gather(i, b).wait()

            @pl.when(i + 1 < NCH)
            def _():
                idx_copy(i + 1, 1 - b).wait()
                gather(i + 1, 1 - b).start()

            @pl.when(i + 2 < NCH)
            def _():
                idx_copy(i + 2, b).start()

            @pl.when(i >= 2)
            def _():
                out_copy(i - 2, b).wait()

            def racc(r, cr):
                def facc(j, accs):
                    a0, a1, a2, a3 = accs
                    base = r * F + j * 20
                    for t in range(5):
                        q = base + t * 4
                        a0 = a0 + rows_v.at[b][q, :]
                        a1 = a1 + rows_v.at[b][q + 1, :]
                        a2 = a2 + rows_v.at[b][q + 2, :]
                        a3 = a3 + rows_v.at[b][q + 3, :]
                    return (a0, a1, a2, a3)
                z = jnp.zeros((D,), jnp.float32)
                a0, a1, a2, a3 = lax.fori_loop(0, F // 20, facc, (z, z, z, z))
                out_v.at[b][r, :] = (a0 + a1) + (a2 + a3)
                return cr

            lax.fori_loop(0, CB, racc, 0)
            out_copy(i, b).start()
        return carry

    lax.fori_loop(0, NCH // 2, outer, 0)
    out_copy(NCH - 2, 0).wait()
    out_copy(NCH - 1, 1).wait()


_sc_pool = functools.partial(
    pl.kernel,
    out_type=jax.ShapeDtypeStruct((BATCH, D), jnp.float32),
    mesh=plsc.VectorSubcoreMesh(core_axis_name="c", subcore_axis_name="s"),
    scratch_types=[
        pltpu.VMEM((2, FLAT), jnp.int32),
        pltpu.VMEM((2, FLAT, D), jnp.float32),
        pltpu.VMEM((2, CB, D), jnp.float32),
        pltpu.SemaphoreType.DMA((2,)),
        pltpu.SemaphoreType.DMA((2,)),
        pltpu.SemaphoreType.DMA((2,)),
    ],
    compiler_params=pltpu.CompilerParams(use_tc_tiling_on_sc=False),
)(_sc_pool_body)


BLK = 2048


def _mlp_body(h_ref, w1_ref, b1_ref, w2_ref, b2_ref, wo_ref, bo_ref, o_ref):
    h = h_ref[...]
    z = jnp.dot(h, w1_ref[...], preferred_element_type=jnp.float32) + b1_ref[...]
    z = jnp.dot(z, w2_ref[...], preferred_element_type=jnp.float32) + b2_ref[...]
    z = jnp.dot(z, wo_ref[...], preferred_element_type=jnp.float32) + bo_ref[...]
    o_ref[...] = jax.nn.sigmoid(z)


def _mlp(pooled, W1, b1, W2, b2, Wout, bout):
    h1, h2, h3 = W1.shape[0], W2.shape[0], Wout.shape[0]
    ncls = Wout.shape[1]
    full = lambda shape: pl.BlockSpec(shape, lambda i: (0, 0))
    return pl.pallas_call(
        _mlp_body,
        grid=(BATCH // BLK,),
        in_specs=[
            pl.BlockSpec((BLK, h1), lambda i: (i, 0)),
            full((h1, h2)), full((1, h2)),
            full((h2, h3)), full((1, h3)),
            full((h3, ncls)), full((1, ncls)),
        ],
        out_specs=pl.BlockSpec((BLK, ncls), lambda i: (i, 0)),
        out_shape=jax.ShapeDtypeStruct((BATCH, ncls), jnp.float32),
    )(pooled, W1, b1.reshape(1, -1), W2, b2.reshape(1, -1),
      Wout, bout.reshape(1, -1))


def kernel(x, emb, W1, b1, W2, b2, Wout, bout):
    pooled = _sc_pool(x.reshape(-1), emb)
    return _mlp(pooled, W1, b1, W2, b2, Wout, bout)


# speedup vs baseline: 52.0592x; 1.0551x over previous
import functools

import jax
import jax.numpy as jnp
from jax import lax
from jax.experimental import pallas as pl
from jax.experimental.pallas import tpu as pltpu
from jax.experimental.pallas import tpu_sc as plsc

BATCH = 16384
F = 100
D = 16
NCLS = 2
NC, NS = 2, 16
NW = NC * NS
PER_W = BATCH // NW
CB = 32
NCH = PER_W // CB
FLAT = CB * F
def _hw_body(w1_ref, b1_ref, w2_ref, b2_ref, wo_ref, bo_ref, o_ref):
    f32 = jnp.float32
    mT = lax.dot_general(wo_ref[...], w2_ref[...], (((0,), (1,)), ((), ())),
                         preferred_element_type=f32, precision=lax.Precision.HIGHEST)
    weffT = lax.dot_general(mT, w1_ref[...], (((1,), (1,)), ((), ())),
                            preferred_element_type=f32, precision=lax.Precision.HIGHEST)
    beff = (lax.dot_general(b1_ref[...], mT, (((1,), (1,)), ((), ())),
                            preferred_element_type=f32,
                            precision=lax.Precision.HIGHEST)
            + jnp.dot(b2_ref[...], wo_ref[...], preferred_element_type=f32,
                      precision=lax.Precision.HIGHEST)
            + bo_ref[...])
    row2 = jnp.concatenate([beff, jnp.zeros((1, D - NCLS), f32)], axis=1)
    o_ref[...] = jnp.concatenate([weffT, row2, jnp.zeros((5, D), f32)], axis=0)


def _head_weights(W1, b1, W2, b2, Wout, bout):
    return pl.pallas_call(
        _hw_body,
        out_shape=jax.ShapeDtypeStruct((8, D), jnp.float32),
    )(W1, b1.reshape(1, -1), W2, b2.reshape(1, -1), Wout, bout.reshape(1, -1))


def _sc_body(x_hbm, emb_hbm, hw_hbm, out_hbm,
             idx_v, rows_v, pool_v, out_v, hw_v, gsem, isem, osem):
    c = lax.axis_index("c")
    s = lax.axis_index("s")
    wid = s * NC + c
    row0 = wid * PER_W

    pltpu.sync_copy(hw_hbm, hw_v)

    def idx_copy(i, buf):
        return pltpu.make_async_copy(
            x_hbm.at[pl.ds((row0 + i * CB) * F, FLAT)], idx_v.at[buf],
            isem.at[buf])

    def gather(i, buf):
        return pltpu.make_async_copy(
            emb_hbm.at[idx_v.at[buf]], rows_v.at[buf], gsem.at[buf])

    def out_copy(i, buf):
        return pltpu.make_async_copy(
            out_v.at[buf], out_hbm.at[pl.ds(row0 + i * CB, CB)], osem.at[buf])

    pltpu.sync_copy(x_hbm.at[pl.ds(row0 * F, FLAT)], idx_v.at[0])
    gather(0, 0).start()
    idx_copy(1, 1).start()

    lanes = jnp.arange(16, dtype=jnp.int32)
    wvA = hw_v[0, :]
    wvB = hw_v[1, :]
    wvC = hw_v[2, :]

    def outer(i0, carry):
        for b in range(2):
            i = i0 * 2 + b
            gather(i, b).wait()

            @pl.when(i + 1 < NCH)
            def _():
                idx_copy(i + 1, 1 - b).wait()
                gather(i + 1, 1 - b).start()

            @pl.when(i + 2 < NCH)
            def _():
                idx_copy(i + 2, b).start()

            @pl.when(i >= 2)
            def _():
                out_copy(i - 2, b).wait()

            def racc(r, cr):
                def facc(j, accs):
                    a0, a1, a2, a3 = accs
                    base = r * F + j * 20
                    for t in range(5):
                        q = base + t * 4
                        a0 = a0 + rows_v.at[b][q, :]
                        a1 = a1 + rows_v.at[b][q + 1, :]
                        a2 = a2 + rows_v.at[b][q + 2, :]
                        a3 = a3 + rows_v.at[b][q + 3, :]
                    return (a0, a1, a2, a3)
                z = jnp.zeros((D,), jnp.float32)
                a0, a1, a2, a3 = lax.fori_loop(0, F // 20, facc, (z, z, z, z))
                pool_v[r, :] = (a0 + a1) + (a2 + a3)
                return cr

            lax.fori_loop(0, CB, racc, 0)

            for g in range(CB // 16):
                r_idx = lanes + g * 16
                z0 = jnp.full((16,), wvC[0], jnp.float32)
                z1 = jnp.full((16,), wvC[1], jnp.float32)
                for d in range(D):
                    col = plsc.load_gather(
                        pool_v, [r_idx, jnp.full((16,), d, jnp.int32)])
                    z0 = z0 + col * wvA[d]
                    z1 = z1 + col * wvB[d]
                s0 = 1.0 / (1.0 + jnp.exp(-z0))
                s1 = 1.0 / (1.0 + jnp.exp(-z1))
                plsc.store_scatter(
                    out_v.at[b], [r_idx, jnp.full((16,), 0, jnp.int32)], s0)
                plsc.store_scatter(
                    out_v.at[b], [r_idx, jnp.full((16,), 1, jnp.int32)], s1)

            out_copy(i, b).start()
        return carry

    lax.fori_loop(0, NCH // 2, outer, 0)
    out_copy(NCH - 2, 0).wait()
    out_copy(NCH - 1, 1).wait()


_sc_main = functools.partial(
    pl.kernel,
    out_type=jax.ShapeDtypeStruct((BATCH, NCLS), jnp.float32),
    mesh=plsc.VectorSubcoreMesh(core_axis_name="c", subcore_axis_name="s"),
    scratch_types=[
        pltpu.VMEM((2, FLAT), jnp.int32),
        pltpu.VMEM((2, FLAT, D), jnp.float32),
        pltpu.VMEM((CB, D), jnp.float32),
        pltpu.VMEM((2, CB, NCLS), jnp.float32),
        pltpu.VMEM((8, D), jnp.float32),
        pltpu.SemaphoreType.DMA((2,)),
        pltpu.SemaphoreType.DMA((2,)),
        pltpu.SemaphoreType.DMA((2,)),
    ],
    compiler_params=pltpu.CompilerParams(use_tc_tiling_on_sc=False,
                                         needs_layout_passes=False),
)(_sc_body)


def kernel(x, emb, W1, b1, W2, b2, Wout, bout):
    hw = _head_weights(W1, b1, W2, b2, Wout, bout)
    return _sc_main(x.reshape(-1), emb, hw)
